# baseline (speedup 1.0000x reference)
import numpy as np
import jax
import jax.numpy as jnp
from jax.experimental import pallas as pl
from jax.experimental.pallas import tpu as pltpu

_N_BP = 4
_WO = _N_BP - 1
_VO = 2


def _consts(T):
    bp = (np.arange(_N_BP, dtype=np.float64) * (float(T) / _WO)).astype(np.float32)
    klo = bp[:-1]
    khi = bp[1:]
    dk = khi - klo
    c_lin = dk.astype(np.float64)
    c_const = 0.5 * c_lin * c_lin - c_lin * khi.astype(np.float64)
    return (
        [float(a) for a in klo],
        [float(a) for a in khi],
        [float(a) for a in dk],
        [float(a) for a in c_const],
    )


def _saaf_body(klo, khi, dk, c0, vt_ref, wt_ref, x_ref, o_ref):
    v0 = vt_ref[0:1, :]
    v1 = vt_ref[1:2, :]
    ws = [wt_ref[k : k + 1, :] for k in range(_WO)]

    a1 = v1
    a0 = v0
    for k in range(_WO):
        a1 = a1 + dk[k] * ws[k]
        a0 = a0 + c0[k] * ws[k]

    x = x_ref[...]
    acc = a0 + a1 * x
    for k in range(_WO):
        t = x - khi[k]
        m = (x > klo[k]) & (x < khi[k])
        acc = acc + jnp.where(m, 0.5 * ws[k], 0.0) * (t * t)
    o_ref[...] = acc


def kernel(x, v, w):
    B, T, C = x.shape
    N = T * C
    xf = x.reshape(B, N)
    vt = v.transpose(1, 2, 0).reshape(_VO, N)
    wt = w.transpose(1, 2, 0).reshape(_WO, N)

    klo, khi, dk, c0 = _consts(T)
    NB = 4096
    body = lambda vr, wr, xr, orf: _saaf_body(klo, khi, dk, c0, vr, wr, xr, orf)
    body.__name__ = "saaf_fused"

    out = pl.pallas_call(
        body,
        grid=(N // NB,),
        in_specs=[
            pl.BlockSpec((_VO, NB), lambda i: (0, i)),
            pl.BlockSpec((_WO, NB), lambda i: (0, i)),
            pl.BlockSpec((B, NB), lambda i: (0, i)),
        ],
        out_specs=pl.BlockSpec((B, NB), lambda i: (0, i)),
        out_shape=jax.ShapeDtypeStruct((B, N), jnp.float32),
        compiler_params=pltpu.CompilerParams(
            dimension_semantics=("arbitrary",),
        ),
    )(vt, wt, xf)
    return out.reshape(B, T, C)


# speedup vs baseline: 1.5086x; 1.5086x over previous
import numpy as np
import jax
import jax.numpy as jnp
from jax.experimental import pallas as pl
from jax.experimental.pallas import tpu as pltpu

_N_BP = 4
_WO = _N_BP - 1
_VO = 2


def _consts(T):
    bp = (np.arange(_N_BP, dtype=np.float64) * (float(T) / _WO)).astype(np.float32)
    klo = bp[:-1]
    khi = bp[1:]
    dk = khi - klo
    c_lin = dk.astype(np.float64)
    c_const = 0.5 * c_lin * c_lin - c_lin * khi.astype(np.float64)
    return (
        [float(a) for a in klo],
        [float(a) for a in khi],
        [float(a) for a in dk],
        [float(a) for a in c_const],
    )


def _saaf_body(klo, khi, dk, c0, B, v_ref, w_ref, x_ref, o_ref):
    C = v_ref.shape[0]
    TB = v_ref.shape[2]
    v0 = v_ref[:, 0, :]
    v1 = v_ref[:, 1, :]
    ws = [w_ref[:, k, :] for k in range(_WO)]

    a1 = v1
    a0 = v0
    for k in range(_WO):
        a1 = a1 + dk[k] * ws[k]
        a0 = a0 + c0[k] * ws[k]
    hws = [0.5 * ws[k] for k in range(_WO)]

    a0r = pltpu.repeat(a0, B, axis=1)
    a1r = pltpu.repeat(a1, B, axis=1)
    hwr = [pltpu.repeat(h, B, axis=1) for h in hws]

    eye = jnp.eye(C, dtype=jnp.float32)
    xm = x_ref[...].reshape(B * TB, C)
    x = jax.lax.dot_general(
        eye, xm, (((1,), (1,)), ((), ())),
        preferred_element_type=jnp.float32,
    )

    acc = a0r + a1r * x
    for k in range(_WO):
        t = x - khi[k]
        m = (x > klo[k]) & (x < khi[k])
        acc = acc + jnp.where(m, hwr[k], 0.0) * (t * t)

    om = jax.lax.dot_general(
        acc, eye, (((0,), (0,)), ((), ())),
        preferred_element_type=jnp.float32,
    )
    o_ref[...] = om.reshape(B, TB, C)


def kernel(x, v, w):
    B, T, C = x.shape
    klo, khi, dk, c0 = _consts(T)

    TB = 256
    n_t = T // TB

    body = lambda vr, wr, xr, orf: _saaf_body(klo, khi, dk, c0, B, vr, wr, xr, orf)
    body.__name__ = "saaf_fused"

    return pl.pallas_call(
        body,
        grid=(n_t,),
        in_specs=[
            pl.BlockSpec((C, _VO, TB), lambda i: (0, 0, i)),
            pl.BlockSpec((C, _WO, TB), lambda i: (0, 0, i)),
            pl.BlockSpec((B, TB, C), lambda i: (0, i, 0)),
        ],
        out_specs=pl.BlockSpec((B, TB, C), lambda i: (0, i, 0)),
        out_shape=jax.ShapeDtypeStruct((B, T, C), jnp.float32),
        compiler_params=pltpu.CompilerParams(
            dimension_semantics=("arbitrary",),
        ),
    )(v, w, x)
